# baseline (speedup 1.0000x reference)
import functools
import jax
import jax.numpy as jnp
from jax import lax
from jax.experimental import pallas as pl
from jax.experimental.pallas import tpu as pltpu
from jax.experimental.pallas import tpu_sc as plsc

_N = 10000
_E = 320000
_F_IN = 128
_HID = 16
_C = 7

_NC = 2
_NS = 16
_CH = 128
_K = 8
_NCA = 1
_G0 = 20
_G1 = 0
_GMAX = max(_G0, _G1)
_NGRP = _NS * (_G0 + _G1)
_NCHK = _NGRP * _K
_EPAD = _NCHK * _CH
_CALLOC = _NCHK + _GMAX * _K
_NPAD = 10240
_RPT = _NPAD // _NS

def _tile_work(c, s):
    if _NCA == 1:
        return _G0, s * (_G0 * _K)
    ngrp = jnp.where(c == 0, _G0, _G1)
    base = jnp.where(c == 0, s * (_G0 * _K),
                     _NS * (_G0 * _K) + s * (_G1 * _K))
    return ngrp, base


def _deg_body(dst_hbm, out_hbm, accd, idx_d, ones_v, tmp_v, sem):
    c = lax.axis_index("c")
    s = lax.axis_index("s")
    ngrp, base = _tile_work(c, s)

    pltpu.sync_copy(dst_hbm.at[pl.ds(base, _GMAX * _K)], idx_d)

    def _o(i, _):
        ones_v[pl.ds(i * 16, 16)] = jnp.ones((16,), jnp.float32)
        return 0
    lax.fori_loop(0, _K * _CH // 16, _o, 0)
    def _z(i, _):
        tmp_v[pl.ds(i * 16, 16)] = jnp.zeros((16,), jnp.float32)
        return 0
    lax.fori_loop(0, _RPT // 16, _z, 0)

    r0 = s * _RPT
    pltpu.sync_copy(tmp_v, accd.at[pl.ds(r0, _RPT)])
    plsc.subcore_barrier()

    def _grp(t, _):
        for i in range(_K):
            pltpu.async_copy(ones_v.at[pl.ds(0, _CH)],
                             accd.at[idx_d.at[t * _K + i]], sem, add=True)
        pltpu.make_async_copy(ones_v, accd.at[pl.ds(0, _K * _CH)], sem).wait()
        return 0

    lax.fori_loop(0, ngrp, _grp, 0)
    plsc.subcore_barrier()

    pltpu.sync_copy(accd.at[pl.ds(r0, _RPT)], tmp_v)
    pltpu.sync_copy(tmp_v, out_hbm.at[pl.ds(c * _NPAD + r0, _RPT)])


@functools.cache
def _deg_kernel():
    return pl.kernel(
        _deg_body,
        out_type=jax.ShapeDtypeStruct((_NCA * _NPAD,), jnp.float32),
        mesh=plsc.VectorSubcoreMesh(core_axis_name="c", subcore_axis_name="s", num_cores=_NCA),
        compiler_params=pltpu.CompilerParams(use_tc_tiling_on_sc=False),
        scratch_types=[
            pltpu.VMEM_SHARED((_NPAD,), jnp.float32),
            pltpu.VMEM((_GMAX * _K, _CH), jnp.int32),
            pltpu.VMEM((_K * _CH,), jnp.float32),
            pltpu.VMEM((_RPT,), jnp.float32),
            pltpu.SemaphoreType.DMA,
        ],
    )


def _segsum_body(src_hbm, dst_hbm, hs_hbm, out_hbm, acc, idx_s, idx_d, msg0,
                 msg1, tmp_v, gsem0, gsem1, ssem0, ssem1):
    c = lax.axis_index("c")
    s = lax.axis_index("s")
    ngrp, base = _tile_work(c, s)

    pltpu.sync_copy(src_hbm.at[pl.ds(base, _GMAX * _K)], idx_s)
    pltpu.sync_copy(dst_hbm.at[pl.ds(base, _GMAX * _K)], idx_d)

    def _z(i, _):
        tmp_v[i] = jnp.zeros((16,), jnp.float32)
        return 0
    lax.fori_loop(0, _RPT, _z, 0)
    r0 = s * _RPT
    pltpu.sync_copy(tmp_v, acc.at[pl.ds(r0, _RPT)])
    plsc.subcore_barrier()

    def _fire_gathers(g, msg, sem):
        for i in range(_K):
            pltpu.async_copy(hs_hbm.at[idx_s.at[g * _K + i]],
                             msg.at[pl.ds(i * _CH, _CH)], sem)

    def _fire_scatters(g, msg, sem):
        for i in range(_K):
            pltpu.async_copy(msg.at[pl.ds(i * _CH, _CH)],
                             acc.at[idx_d.at[g * _K + i]], sem, add=True)

    def _drain(msg, sem):
        pltpu.make_async_copy(msg, acc.at[pl.ds(0, _K * _CH)], sem).wait()

    _fire_gathers(0, msg0, gsem0)
    _fire_gathers(1, msg1, gsem1)
    _drain(msg0, gsem0)
    _fire_scatters(0, msg0, ssem0)

    def _pair(t, _):
        g = 2 * t + 1
        _drain(msg0, ssem0)
        _fire_gathers(g + 1, msg0, gsem0)
        _drain(msg1, gsem1)
        _fire_scatters(g, msg1, ssem1)
        _drain(msg1, ssem1)
        _fire_gathers(g + 2, msg1, gsem1)
        _drain(msg0, gsem0)
        _fire_scatters(g + 1, msg0, ssem0)
        return 0

    lax.fori_loop(0, ngrp // 2 - 1, _pair, 0)
    _drain(msg0, ssem0)
    _drain(msg1, gsem1)
    _fire_scatters(ngrp - 1, msg1, ssem1)
    _drain(msg1, ssem1)

    plsc.subcore_barrier()
    pltpu.sync_copy(acc.at[pl.ds(r0, _RPT)], tmp_v)
    pltpu.sync_copy(tmp_v, out_hbm.at[pl.ds(c * _NPAD + r0, _RPT)])


@functools.cache
def _segsum_kernel():
    return pl.kernel(
        _segsum_body,
        out_type=jax.ShapeDtypeStruct((_NCA * _NPAD, _HID), jnp.float32),
        mesh=plsc.VectorSubcoreMesh(core_axis_name="c", subcore_axis_name="s", num_cores=_NCA),
        compiler_params=pltpu.CompilerParams(use_tc_tiling_on_sc=False),
        scratch_types=[
            pltpu.VMEM_SHARED((_NPAD, _HID), jnp.float32),
            pltpu.VMEM((_GMAX * _K, _CH), jnp.int32),
            pltpu.VMEM((_GMAX * _K, _CH), jnp.int32),
            pltpu.VMEM((_K * _CH, _HID), jnp.float32),
            pltpu.VMEM((_K * _CH, _HID), jnp.float32),
            pltpu.VMEM((_RPT, _HID), jnp.float32),
            pltpu.SemaphoreType.DMA,
            pltpu.SemaphoreType.DMA,
            pltpu.SemaphoreType.DMA,
            pltpu.SemaphoreType.DMA,
        ],
    )


_BLK = 1024
_NBLK = _NPAD // _BLK


def _dinv(degp_ref):
    deg = degp_ref[0, :] + 1.0
    if _NCA == 2:
        deg = deg + degp_ref[1, :]
    return lax.rsqrt(deg)


def _tc1_body(x_ref, w1_ref, degp_ref, out_ref):
    dinv = _dinv(degp_ref)
    h = jnp.dot(x_ref[...], w1_ref[...], preferred_element_type=jnp.float32)
    out_ref[...] = h * dinv[:, None]


def _tc2_body(*refs):
    (*ps, hs1_ref, degp_ref, b1_ref, w2_ref, out_ref) = refs
    dinv = _dinv(degp_ref)
    acc = sum(p[...] for p in ps) + hs1_ref[...]
    acc = acc * dinv[:, None]
    h2 = jnp.maximum(acc + b1_ref[...], 0.0)
    out_ref[...] = jnp.dot(
        h2, w2_ref[...], preferred_element_type=jnp.float32) * dinv[:, None]


def _tc3_body(*refs):
    (*ps, hs2_ref, degp_ref, b2_ref, out_ref) = refs
    dinv = _dinv(degp_ref)
    acc = sum(p[...] for p in ps) + hs2_ref[...]
    acc = acc * dinv[:, None]
    out_ref[...] = jnp.maximum(acc + b2_ref[...], 0.0)


def _row_spec(off):
    return pl.BlockSpec((_BLK, _HID), lambda i, off=off: (i + off, 0))


def kernel(x, edge_index, W1, b1, W2, b2):
    pad = _CALLOC * _CH - _E
    src = jnp.concatenate(
        [edge_index[0], jnp.zeros((pad,), jnp.int32)]).reshape(_CALLOC, _CH)
    dst = jnp.concatenate(
        [edge_index[1], jnp.full((pad,), _N, jnp.int32)]).reshape(
            _CALLOC, _CH)
    x_p = jnp.pad(x, ((0, _NPAD - _N), (0, 0)))
    w2_p = jnp.pad(W2, ((0, 0), (0, _HID - _C)))
    b1_r = b1.reshape(1, _HID)
    b2_r = jnp.pad(b2, (0, _HID - _C)).reshape(1, _HID)

    degp = _deg_kernel()(dst).reshape(_NCA, _NPAD)

    p_specs = [_row_spec(i * _NBLK) for i in range(_NCA)]
    degp_spec = pl.BlockSpec((_NCA, _BLK), lambda i: (0, i))

    hs1 = pl.pallas_call(
        _tc1_body,
        grid=(_NBLK,),
        in_specs=[
            pl.BlockSpec((_BLK, _F_IN), lambda i: (i, 0)),
            pl.BlockSpec((_F_IN, _HID), lambda i: (0, 0)),
            degp_spec,
        ],
        out_specs=pl.BlockSpec((_BLK, _HID), lambda i: (i, 0)),
        out_shape=jax.ShapeDtypeStruct((_NPAD, _HID), jnp.float32),
    )(x_p, W1, degp)

    acc1 = _segsum_kernel()(src, dst, hs1)

    hs2 = pl.pallas_call(
        _tc2_body,
        grid=(_NBLK,),
        in_specs=p_specs + [
            pl.BlockSpec((_BLK, _HID), lambda i: (i, 0)),
            degp_spec,
            pl.BlockSpec((1, _HID), lambda i: (0, 0)),
            pl.BlockSpec((_HID, _HID), lambda i: (0, 0)),
        ],
        out_specs=pl.BlockSpec((_BLK, _HID), lambda i: (i, 0)),
        out_shape=jax.ShapeDtypeStruct((_NPAD, _HID), jnp.float32),
    )(*([acc1] * _NCA), hs1, degp, b1_r, w2_p)

    acc2 = _segsum_kernel()(src, dst, hs2)

    out = pl.pallas_call(
        _tc3_body,
        grid=(_NBLK,),
        in_specs=p_specs + [
            pl.BlockSpec((_BLK, _HID), lambda i: (i, 0)),
            degp_spec,
            pl.BlockSpec((1, _HID), lambda i: (0, 0)),
        ],
        out_specs=pl.BlockSpec((_BLK, _HID), lambda i: (i, 0)),
        out_shape=jax.ShapeDtypeStruct((_NPAD, _HID), jnp.float32),
    )(*([acc2] * _NCA), hs2, degp, b2_r)

    return out[:_N, :_C]


# speedup vs baseline: 1.1014x; 1.1014x over previous
import functools
import jax
import jax.numpy as jnp
from jax import lax
from jax.experimental import pallas as pl
from jax.experimental.pallas import tpu as pltpu
from jax.experimental.pallas import tpu_sc as plsc

_N = 10000
_E = 320000
_F_IN = 128
_HID = 16
_C = 7

_NC = 2
_NS = 16
_CH = 128
_K = 8
_NCA = 2
_G0 = 12
_G1 = 8
_GMAX = max(_G0, _G1)
_NGRP = _NS * (_G0 + _G1)
_NCHK = _NGRP * _K
_EPAD = _NCHK * _CH
_CALLOC = _NCHK + _GMAX * _K
_NPAD = 10240
_RPT = _NPAD // _NS

def _tile_work(c, s):
    if _NCA == 1:
        return _G0, s * (_G0 * _K)
    ngrp = jnp.where(c == 0, _G0, _G1)
    base = jnp.where(c == 0, s * (_G0 * _K),
                     _NS * (_G0 * _K) + s * (_G1 * _K))
    return ngrp, base


def _preload(c, base, hbm, vmem):
    @pl.when(c == 0)
    def _():
        pltpu.sync_copy(hbm.at[pl.ds(base, _G0 * _K)],
                        vmem.at[pl.ds(0, _G0 * _K)])
    @pl.when(c != 0)
    def _():
        pltpu.sync_copy(hbm.at[pl.ds(base, _G1 * _K)],
                        vmem.at[pl.ds(0, _G1 * _K)])


def _deg_body(dst_hbm, out_hbm, accd, idx_d, ones_v, tmp_v, sem):
    c = lax.axis_index("c")
    s = lax.axis_index("s")
    ngrp, base = _tile_work(c, s)

    _preload(c, base, dst_hbm, idx_d)

    def _o(i, _):
        ones_v[pl.ds(i * 16, 16)] = jnp.ones((16,), jnp.float32)
        return 0
    lax.fori_loop(0, _K * _CH // 16, _o, 0)
    def _z(i, _):
        tmp_v[pl.ds(i * 16, 16)] = jnp.zeros((16,), jnp.float32)
        return 0
    lax.fori_loop(0, _RPT // 16, _z, 0)

    r0 = s * _RPT
    pltpu.sync_copy(tmp_v, accd.at[pl.ds(r0, _RPT)])
    plsc.subcore_barrier()

    def _grp(t, _):
        for i in range(_K):
            pltpu.async_copy(ones_v.at[pl.ds(0, _CH)],
                             accd.at[idx_d.at[t * _K + i]], sem, add=True)
        pltpu.make_async_copy(ones_v, accd.at[pl.ds(0, _K * _CH)], sem).wait()
        return 0

    lax.fori_loop(0, ngrp, _grp, 0)
    plsc.subcore_barrier()

    pltpu.sync_copy(accd.at[pl.ds(r0, _RPT)], tmp_v)
    pltpu.sync_copy(tmp_v, out_hbm.at[pl.ds(c * _NPAD + r0, _RPT)])


@functools.cache
def _deg_kernel():
    return pl.kernel(
        _deg_body,
        out_type=jax.ShapeDtypeStruct((_NCA * _NPAD,), jnp.float32),
        mesh=plsc.VectorSubcoreMesh(core_axis_name="c", subcore_axis_name="s", num_cores=_NCA),
        compiler_params=pltpu.CompilerParams(use_tc_tiling_on_sc=False),
        scratch_types=[
            pltpu.VMEM_SHARED((_NPAD,), jnp.float32),
            pltpu.VMEM((_GMAX * _K, _CH), jnp.int32),
            pltpu.VMEM((_K * _CH,), jnp.float32),
            pltpu.VMEM((_RPT,), jnp.float32),
            pltpu.SemaphoreType.DMA,
        ],
    )


def _segsum_body(src_hbm, dst_hbm, hs_hbm, out_hbm, acc, idx_s, idx_d, msg0,
                 msg1, tmp_v, gsem0, gsem1, ssem0, ssem1):
    c = lax.axis_index("c")
    s = lax.axis_index("s")
    ngrp, base = _tile_work(c, s)

    _preload(c, base, src_hbm, idx_s)
    _preload(c, base, dst_hbm, idx_d)

    def _z(i, _):
        tmp_v[i] = jnp.zeros((16,), jnp.float32)
        return 0
    lax.fori_loop(0, _RPT // 4, _z, 0)
    r0 = s * _RPT
    for q in range(4):
        pltpu.sync_copy(tmp_v.at[pl.ds(0, _RPT // 4)],
                        acc.at[pl.ds(r0 + q * (_RPT // 4), _RPT // 4)])
    plsc.subcore_barrier()

    def _fire_gathers(g, msg, sem):
        for i in range(_K):
            pltpu.async_copy(hs_hbm.at[idx_s.at[g * _K + i]],
                             msg.at[pl.ds(i * _CH, _CH)], sem)

    def _fire_scatters(g, msg, sem):
        for i in range(_K):
            pltpu.async_copy(msg.at[pl.ds(i * _CH, _CH)],
                             acc.at[idx_d.at[g * _K + i]], sem, add=True)

    def _drain(msg, sem):
        pltpu.make_async_copy(msg, acc.at[pl.ds(0, _K * _CH)], sem).wait()

    _fire_gathers(0, msg0, gsem0)
    _fire_gathers(1, msg1, gsem1)
    _drain(msg0, gsem0)
    _fire_scatters(0, msg0, ssem0)

    def _pair(t, _):
        g = 2 * t + 1
        _drain(msg0, ssem0)
        _fire_gathers(g + 1, msg0, gsem0)
        _drain(msg1, gsem1)
        _fire_scatters(g, msg1, ssem1)
        _drain(msg1, ssem1)
        _fire_gathers(g + 2, msg1, gsem1)
        _drain(msg0, gsem0)
        _fire_scatters(g + 1, msg0, ssem0)
        return 0

    lax.fori_loop(0, ngrp // 2 - 1, _pair, 0)
    _drain(msg0, ssem0)
    _drain(msg1, gsem1)
    _fire_scatters(ngrp - 1, msg1, ssem1)
    _drain(msg1, ssem1)

    plsc.subcore_barrier()
    pltpu.sync_copy(acc.at[pl.ds(r0, _RPT)], tmp_v)
    pltpu.sync_copy(tmp_v, out_hbm.at[pl.ds(c * _NPAD + r0, _RPT)])


@functools.cache
def _segsum_kernel():
    return pl.kernel(
        _segsum_body,
        out_type=jax.ShapeDtypeStruct((_NCA * _NPAD, _HID), jnp.float32),
        mesh=plsc.VectorSubcoreMesh(core_axis_name="c", subcore_axis_name="s", num_cores=_NCA),
        compiler_params=pltpu.CompilerParams(use_tc_tiling_on_sc=False),
        scratch_types=[
            pltpu.VMEM_SHARED((_NPAD, _HID), jnp.float32),
            pltpu.VMEM((_GMAX * _K, _CH), jnp.int32),
            pltpu.VMEM((_GMAX * _K, _CH), jnp.int32),
            pltpu.VMEM((_K * _CH, _HID), jnp.float32),
            pltpu.VMEM((_K * _CH, _HID), jnp.float32),
            pltpu.VMEM((_RPT, _HID), jnp.float32),
            pltpu.SemaphoreType.DMA,
            pltpu.SemaphoreType.DMA,
            pltpu.SemaphoreType.DMA,
            pltpu.SemaphoreType.DMA,
        ],
    )


_BLK = 1024
_NBLK = _NPAD // _BLK


def _dinv(degp_ref):
    deg = degp_ref[0, :] + 1.0
    if _NCA == 2:
        deg = deg + degp_ref[1, :]
    return lax.rsqrt(deg)


def _tc1_body(x_ref, w1_ref, degp_ref, out_ref):
    dinv = _dinv(degp_ref)
    h = jnp.dot(x_ref[...], w1_ref[...], preferred_element_type=jnp.float32)
    out_ref[...] = h * dinv[:, None]


def _tc2_body(*refs):
    (*ps, hs1_ref, degp_ref, b1_ref, w2_ref, out_ref) = refs
    dinv = _dinv(degp_ref)
    acc = sum(p[...] for p in ps) + hs1_ref[...]
    acc = acc * dinv[:, None]
    h2 = jnp.maximum(acc + b1_ref[...], 0.0)
    out_ref[...] = jnp.dot(
        h2, w2_ref[...], preferred_element_type=jnp.float32) * dinv[:, None]


def _tc3_body(*refs):
    (*ps, hs2_ref, degp_ref, b2_ref, out_ref) = refs
    dinv = _dinv(degp_ref)
    acc = sum(p[...] for p in ps) + hs2_ref[...]
    acc = acc * dinv[:, None]
    out_ref[...] = jnp.maximum(acc + b2_ref[...], 0.0)


def _row_spec(off):
    return pl.BlockSpec((_BLK, _HID), lambda i, off=off: (i + off, 0))


def kernel(x, edge_index, W1, b1, W2, b2):
    pad = _CALLOC * _CH - _E
    src = jnp.concatenate(
        [edge_index[0], jnp.zeros((pad,), jnp.int32)]).reshape(_CALLOC, _CH)
    dst = jnp.concatenate(
        [edge_index[1], jnp.full((pad,), _N, jnp.int32)]).reshape(
            _CALLOC, _CH)
    x_p = jnp.pad(x, ((0, _NPAD - _N), (0, 0)))
    w2_p = jnp.pad(W2, ((0, 0), (0, _HID - _C)))
    b1_r = b1.reshape(1, _HID)
    b2_r = jnp.pad(b2, (0, _HID - _C)).reshape(1, _HID)

    degp = _deg_kernel()(dst).reshape(_NCA, _NPAD)

    p_specs = [_row_spec(i * _NBLK) for i in range(_NCA)]
    degp_spec = pl.BlockSpec((_NCA, _BLK), lambda i: (0, i))

    hs1 = pl.pallas_call(
        _tc1_body,
        grid=(_NBLK,),
        in_specs=[
            pl.BlockSpec((_BLK, _F_IN), lambda i: (i, 0)),
            pl.BlockSpec((_F_IN, _HID), lambda i: (0, 0)),
            degp_spec,
        ],
        out_specs=pl.BlockSpec((_BLK, _HID), lambda i: (i, 0)),
        out_shape=jax.ShapeDtypeStruct((_NPAD, _HID), jnp.float32),
    )(x_p, W1, degp)

    acc1 = _segsum_kernel()(src, dst, hs1)

    hs2 = pl.pallas_call(
        _tc2_body,
        grid=(_NBLK,),
        in_specs=p_specs + [
            pl.BlockSpec((_BLK, _HID), lambda i: (i, 0)),
            degp_spec,
            pl.BlockSpec((1, _HID), lambda i: (0, 0)),
            pl.BlockSpec((_HID, _HID), lambda i: (0, 0)),
        ],
        out_specs=pl.BlockSpec((_BLK, _HID), lambda i: (i, 0)),
        out_shape=jax.ShapeDtypeStruct((_NPAD, _HID), jnp.float32),
    )(*([acc1] * _NCA), hs1, degp, b1_r, w2_p)

    acc2 = _segsum_kernel()(src, dst, hs2)

    out = pl.pallas_call(
        _tc3_body,
        grid=(_NBLK,),
        in_specs=p_specs + [
            pl.BlockSpec((_BLK, _HID), lambda i: (i, 0)),
            degp_spec,
            pl.BlockSpec((1, _HID), lambda i: (0, 0)),
        ],
        out_specs=pl.BlockSpec((_BLK, _HID), lambda i: (i, 0)),
        out_shape=jax.ShapeDtypeStruct((_NPAD, _HID), jnp.float32),
    )(*([acc2] * _NCA), hs2, degp, b2_r)

    return out[:_N, :_C]


# speedup vs baseline: 1.2472x; 1.1324x over previous
import functools
import jax
import jax.numpy as jnp
from jax import lax
from jax.experimental import pallas as pl
from jax.experimental.pallas import tpu as pltpu
from jax.experimental.pallas import tpu_sc as plsc

_N = 10000
_E = 320000
_F_IN = 128
_HID = 16
_C = 7

_NC = 2
_NS = 16
_CH = 128
_K = 8
_NCA = 2
_G0 = 14
_G1 = 6
_GMAX = max(_G0, _G1)
_NGRP = _NS * (_G0 + _G1)
_NCHK = _NGRP * _K
_EPAD = _NCHK * _CH
_CALLOC = _NCHK + _GMAX * _K
_NPAD = 10240
_RPT = _NPAD // _NS

def _tile_work(c, s):
    if _NCA == 1:
        return _G0, s * (_G0 * _K)
    ngrp = jnp.where(c == 0, _G0, _G1)
    base = jnp.where(c == 0, s * (_G0 * _K),
                     _NS * (_G0 * _K) + s * (_G1 * _K))
    return ngrp, base


def _preload(c, base, hbm, vmem):
    @pl.when(c == 0)
    def _():
        pltpu.sync_copy(hbm.at[pl.ds(base, _G0 * _K)],
                        vmem.at[pl.ds(0, _G0 * _K)])
    @pl.when(c != 0)
    def _():
        pltpu.sync_copy(hbm.at[pl.ds(base, _G1 * _K)],
                        vmem.at[pl.ds(0, _G1 * _K)])


def _deg_body(dst_hbm, out_hbm, accd, idx_d, ones_v, tmp_v, sem):
    c = lax.axis_index("c")
    s = lax.axis_index("s")
    ngrp, base = _tile_work(c, s)

    _preload(c, base, dst_hbm, idx_d)

    def _o(i, _):
        ones_v[pl.ds(i * 16, 16)] = jnp.ones((16,), jnp.float32)
        return 0
    lax.fori_loop(0, _K * _CH // 16, _o, 0)
    def _z(i, _):
        tmp_v[pl.ds(i * 16, 16)] = jnp.zeros((16,), jnp.float32)
        return 0
    lax.fori_loop(0, _RPT // 16, _z, 0)

    r0 = s * _RPT
    pltpu.sync_copy(tmp_v, accd.at[pl.ds(r0, _RPT)])
    plsc.subcore_barrier()

    def _grp(t, _):
        for i in range(_K):
            pltpu.async_copy(ones_v.at[pl.ds(0, _CH)],
                             accd.at[idx_d.at[t * _K + i]], sem, add=True)
        pltpu.make_async_copy(ones_v, accd.at[pl.ds(0, _K * _CH)], sem).wait()
        return 0

    lax.fori_loop(0, ngrp, _grp, 0)
    plsc.subcore_barrier()

    pltpu.sync_copy(accd.at[pl.ds(r0, _RPT)], tmp_v)
    pltpu.sync_copy(tmp_v, out_hbm.at[pl.ds(c * _NPAD + r0, _RPT)])


@functools.cache
def _deg_kernel():
    return pl.kernel(
        _deg_body,
        out_type=jax.ShapeDtypeStruct((_NCA * _NPAD,), jnp.float32),
        mesh=plsc.VectorSubcoreMesh(core_axis_name="c", subcore_axis_name="s", num_cores=_NCA),
        compiler_params=pltpu.CompilerParams(use_tc_tiling_on_sc=False),
        scratch_types=[
            pltpu.VMEM_SHARED((_NPAD,), jnp.float32),
            pltpu.VMEM((_GMAX * _K, _CH), jnp.int32),
            pltpu.VMEM((_K * _CH,), jnp.float32),
            pltpu.VMEM((_RPT,), jnp.float32),
            pltpu.SemaphoreType.DMA,
        ],
    )


def _segsum_body(src_hbm, dst_hbm, hs_hbm, out_hbm, acc, idx_s, idx_d, msg0,
                 msg1, tmp_v, gsem0, gsem1, ssem0, ssem1):
    c = lax.axis_index("c")
    s = lax.axis_index("s")
    ngrp, base = _tile_work(c, s)

    _preload(c, base, src_hbm, idx_s)
    _preload(c, base, dst_hbm, idx_d)

    def _z(i, _):
        tmp_v[i] = jnp.zeros((16,), jnp.float32)
        return 0
    lax.fori_loop(0, _RPT // 4, _z, 0)
    r0 = s * _RPT
    for q in range(4):
        pltpu.sync_copy(tmp_v.at[pl.ds(0, _RPT // 4)],
                        acc.at[pl.ds(r0 + q * (_RPT // 4), _RPT // 4)])
    plsc.subcore_barrier()

    def _fire_gathers(g, msg, sem):
        for i in range(_K):
            pltpu.async_copy(hs_hbm.at[idx_s.at[g * _K + i]],
                             msg.at[pl.ds(i * _CH, _CH)], sem)

    def _fire_scatters(g, msg, sem):
        for i in range(_K):
            pltpu.async_copy(msg.at[pl.ds(i * _CH, _CH)],
                             acc.at[idx_d.at[g * _K + i]], sem, add=True)

    def _drain(msg, sem):
        pltpu.make_async_copy(msg, acc.at[pl.ds(0, _K * _CH)], sem).wait()

    _fire_gathers(0, msg0, gsem0)
    _fire_gathers(1, msg1, gsem1)
    _drain(msg0, gsem0)
    _fire_scatters(0, msg0, ssem0)

    def _pair(t, _):
        g = 2 * t + 1
        _drain(msg0, ssem0)
        _fire_gathers(g + 1, msg0, gsem0)
        _drain(msg1, gsem1)
        _fire_scatters(g, msg1, ssem1)
        _drain(msg1, ssem1)
        _fire_gathers(g + 2, msg1, gsem1)
        _drain(msg0, gsem0)
        _fire_scatters(g + 1, msg0, ssem0)
        return 0

    lax.fori_loop(0, ngrp // 2 - 1, _pair, 0)
    _drain(msg0, ssem0)
    _drain(msg1, gsem1)
    _fire_scatters(ngrp - 1, msg1, ssem1)
    _drain(msg1, ssem1)

    plsc.subcore_barrier()
    pltpu.sync_copy(acc.at[pl.ds(r0, _RPT)], tmp_v)
    pltpu.sync_copy(tmp_v, out_hbm.at[pl.ds(c * _NPAD + r0, _RPT)])


@functools.cache
def _segsum_kernel():
    return pl.kernel(
        _segsum_body,
        out_type=jax.ShapeDtypeStruct((_NCA * _NPAD, _HID), jnp.float32),
        mesh=plsc.VectorSubcoreMesh(core_axis_name="c", subcore_axis_name="s", num_cores=_NCA),
        compiler_params=pltpu.CompilerParams(use_tc_tiling_on_sc=False),
        scratch_types=[
            pltpu.VMEM_SHARED((_NPAD, _HID), jnp.float32),
            pltpu.VMEM((_GMAX * _K, _CH), jnp.int32),
            pltpu.VMEM((_GMAX * _K, _CH), jnp.int32),
            pltpu.VMEM((_K * _CH, _HID), jnp.float32),
            pltpu.VMEM((_K * _CH, _HID), jnp.float32),
            pltpu.VMEM((_RPT, _HID), jnp.float32),
            pltpu.SemaphoreType.DMA,
            pltpu.SemaphoreType.DMA,
            pltpu.SemaphoreType.DMA,
            pltpu.SemaphoreType.DMA,
        ],
    )


_BLK = 1024
_NBLK = _NPAD // _BLK


def _dinv(degp_ref):
    deg = degp_ref[0, :] + 1.0
    if _NCA == 2:
        deg = deg + degp_ref[1, :]
    return lax.rsqrt(deg)


def _tc1_body(x_ref, w1_ref, degp_ref, out_ref):
    dinv = _dinv(degp_ref)
    h = jnp.dot(x_ref[...], w1_ref[...], preferred_element_type=jnp.float32)
    out_ref[...] = h * dinv[:, None]


def _tc2_body(*refs):
    (*ps, hs1_ref, degp_ref, b1_ref, w2_ref, out_ref) = refs
    dinv = _dinv(degp_ref)
    acc = sum(p[...] for p in ps) + hs1_ref[...]
    acc = acc * dinv[:, None]
    h2 = jnp.maximum(acc + b1_ref[...], 0.0)
    out_ref[...] = jnp.dot(
        h2, w2_ref[...], preferred_element_type=jnp.float32) * dinv[:, None]


def _tc3_body(*refs):
    (*ps, hs2_ref, degp_ref, b2_ref, out_ref) = refs
    dinv = _dinv(degp_ref)
    acc = sum(p[...] for p in ps) + hs2_ref[...]
    acc = acc * dinv[:, None]
    out_ref[...] = jnp.maximum(acc + b2_ref[...], 0.0)


def _row_spec(off):
    return pl.BlockSpec((_BLK, _HID), lambda i, off=off: (i + off, 0))


def kernel(x, edge_index, W1, b1, W2, b2):
    pad = _CALLOC * _CH - _E
    src = jnp.concatenate(
        [edge_index[0], jnp.zeros((pad,), jnp.int32)]).reshape(_CALLOC, _CH)
    dst = jnp.concatenate(
        [edge_index[1], jnp.full((pad,), _N, jnp.int32)]).reshape(
            _CALLOC, _CH)
    x_p = jnp.pad(x, ((0, _NPAD - _N), (0, 0)))
    w2_p = jnp.pad(W2, ((0, 0), (0, _HID - _C)))
    b1_r = b1.reshape(1, _HID)
    b2_r = jnp.pad(b2, (0, _HID - _C)).reshape(1, _HID)

    degp = _deg_kernel()(dst).reshape(_NCA, _NPAD)

    p_specs = [_row_spec(i * _NBLK) for i in range(_NCA)]
    degp_spec = pl.BlockSpec((_NCA, _BLK), lambda i: (0, i))

    hs1 = pl.pallas_call(
        _tc1_body,
        grid=(_NBLK,),
        in_specs=[
            pl.BlockSpec((_BLK, _F_IN), lambda i: (i, 0)),
            pl.BlockSpec((_F_IN, _HID), lambda i: (0, 0)),
            degp_spec,
        ],
        out_specs=pl.BlockSpec((_BLK, _HID), lambda i: (i, 0)),
        out_shape=jax.ShapeDtypeStruct((_NPAD, _HID), jnp.float32),
    )(x_p, W1, degp)

    acc1 = _segsum_kernel()(src, dst, hs1)

    hs2 = pl.pallas_call(
        _tc2_body,
        grid=(_NBLK,),
        in_specs=p_specs + [
            pl.BlockSpec((_BLK, _HID), lambda i: (i, 0)),
            degp_spec,
            pl.BlockSpec((1, _HID), lambda i: (0, 0)),
            pl.BlockSpec((_HID, _HID), lambda i: (0, 0)),
        ],
        out_specs=pl.BlockSpec((_BLK, _HID), lambda i: (i, 0)),
        out_shape=jax.ShapeDtypeStruct((_NPAD, _HID), jnp.float32),
    )(*([acc1] * _NCA), hs1, degp, b1_r, w2_p)

    acc2 = _segsum_kernel()(src, dst, hs2)

    out = pl.pallas_call(
        _tc3_body,
        grid=(_NBLK,),
        in_specs=p_specs + [
            pl.BlockSpec((_BLK, _HID), lambda i: (i, 0)),
            degp_spec,
            pl.BlockSpec((1, _HID), lambda i: (0, 0)),
        ],
        out_specs=pl.BlockSpec((_BLK, _HID), lambda i: (i, 0)),
        out_shape=jax.ShapeDtypeStruct((_NPAD, _HID), jnp.float32),
    )(*([acc2] * _NCA), hs2, degp, b2_r)

    return out[:_N, :_C]


# speedup vs baseline: 1.6216x; 1.3002x over previous
import functools
import jax
import jax.numpy as jnp
from jax import lax
from jax.experimental import pallas as pl
from jax.experimental.pallas import tpu as pltpu
from jax.experimental.pallas import tpu_sc as plsc

_N = 10000
_E = 320000
_F_IN = 128
_HID = 16
_C = 7

_NC = 2
_NS = 16
_CH = 128
_K = 8
_NCA = 2
_G0 = 14
_G1 = 6
_GMAX = max(_G0, _G1)
_NGRP = _NS * (_G0 + _G1)
_NCHK = _NGRP * _K
_EPAD = _NCHK * _CH
_CALLOC = _NCHK + _GMAX * _K
_NPAD = 10240
_RPT = _NPAD // _NS

def _tile_work(c, s):
    if _NCA == 1:
        return _G0, s * (_G0 * _K)
    ngrp = jnp.where(c == 0, _G0, _G1)
    base = jnp.where(c == 0, s * (_G0 * _K),
                     _NS * (_G0 * _K) + s * (_G1 * _K))
    return ngrp, base


def _preload(c, base, hbm, vmem):
    @pl.when(c == 0)
    def _():
        pltpu.sync_copy(hbm.at[pl.ds(base, _G0 * _K)],
                        vmem.at[pl.ds(0, _G0 * _K)])
    @pl.when(c != 0)
    def _():
        pltpu.sync_copy(hbm.at[pl.ds(base, _G1 * _K)],
                        vmem.at[pl.ds(0, _G1 * _K)])


def _deg_body(dst_hbm, out_hbm, accd, idx_d, ones_v, tmp_v, sem):
    c = lax.axis_index("c")
    s = lax.axis_index("s")
    ngrp, base = _tile_work(c, s)

    _preload(c, base, dst_hbm, idx_d)

    def _o(i, _):
        ones_v[pl.ds(i * 16, 16)] = jnp.ones((16,), jnp.float32)
        return 0
    lax.fori_loop(0, _K * _CH // 16, _o, 0)
    def _z(i, _):
        tmp_v[pl.ds(i * 16, 16)] = jnp.zeros((16,), jnp.float32)
        return 0
    lax.fori_loop(0, _RPT // 16, _z, 0)

    r0 = s * _RPT
    pltpu.sync_copy(tmp_v, accd.at[pl.ds(r0, _RPT)])
    plsc.subcore_barrier()

    def _grp(t, _):
        for i in range(_K):
            pltpu.async_copy(ones_v.at[pl.ds(0, _CH)],
                             accd.at[idx_d.at[t * _K + i]], sem, add=True)
        pltpu.make_async_copy(ones_v, accd.at[pl.ds(0, _K * _CH)], sem).wait()
        return 0

    lax.fori_loop(0, ngrp, _grp, 0)
    plsc.subcore_barrier()

    pltpu.sync_copy(accd.at[pl.ds(r0, _RPT)], tmp_v)
    pltpu.sync_copy(tmp_v, out_hbm.at[pl.ds(c * _NPAD + r0, _RPT)])


@functools.cache
def _deg_kernel():
    return pl.kernel(
        _deg_body,
        out_type=jax.ShapeDtypeStruct((_NCA * _NPAD,), jnp.float32),
        mesh=plsc.VectorSubcoreMesh(core_axis_name="c", subcore_axis_name="s", num_cores=_NCA),
        compiler_params=pltpu.CompilerParams(use_tc_tiling_on_sc=False),
        scratch_types=[
            pltpu.VMEM_SHARED((_NPAD,), jnp.float32),
            pltpu.VMEM((_GMAX * _K, _CH), jnp.int32),
            pltpu.VMEM((_K * _CH,), jnp.float32),
            pltpu.VMEM((_RPT,), jnp.float32),
            pltpu.SemaphoreType.DMA,
        ],
    )


def _segsum_body(src_hbm, dst_hbm, hs_hbm, out_hbm, acc, hs_sh, idx_s, idx_d,
                 msg0, msg1, tmp_v, gsem0, gsem1, ssem0, ssem1):
    c = lax.axis_index("c")
    s = lax.axis_index("s")
    ngrp, base = _tile_work(c, s)

    _preload(c, base, src_hbm, idx_s)
    _preload(c, base, dst_hbm, idx_d)

    r0 = s * _RPT
    pltpu.sync_copy(hs_hbm.at[pl.ds(r0, _RPT)], tmp_v)
    pltpu.sync_copy(tmp_v, hs_sh.at[pl.ds(r0, _RPT)])

    def _z(i, _):
        tmp_v[i] = jnp.zeros((16,), jnp.float32)
        return 0
    lax.fori_loop(0, _RPT // 4, _z, 0)
    for q in range(4):
        pltpu.sync_copy(tmp_v.at[pl.ds(0, _RPT // 4)],
                        acc.at[pl.ds(r0 + q * (_RPT // 4), _RPT // 4)])
    plsc.subcore_barrier()

    def _fire_gathers(g, msg, sem):
        for i in range(_K):
            pltpu.async_copy(hs_sh.at[idx_s.at[g * _K + i]],
                             msg.at[pl.ds(i * _CH, _CH)], sem)

    def _fire_scatters(g, msg, sem):
        for i in range(_K):
            pltpu.async_copy(msg.at[pl.ds(i * _CH, _CH)],
                             acc.at[idx_d.at[g * _K + i]], sem, add=True)

    def _drain(msg, sem):
        pltpu.make_async_copy(msg, acc.at[pl.ds(0, _K * _CH)], sem).wait()

    _fire_gathers(0, msg0, gsem0)
    _fire_gathers(1, msg1, gsem1)
    _drain(msg0, gsem0)
    _fire_scatters(0, msg0, ssem0)

    def _pair(t, _):
        g = 2 * t + 1
        _drain(msg0, ssem0)
        _fire_gathers(g + 1, msg0, gsem0)
        _drain(msg1, gsem1)
        _fire_scatters(g, msg1, ssem1)
        _drain(msg1, ssem1)
        _fire_gathers(g + 2, msg1, gsem1)
        _drain(msg0, gsem0)
        _fire_scatters(g + 1, msg0, ssem0)
        return 0

    lax.fori_loop(0, ngrp // 2 - 1, _pair, 0)
    _drain(msg0, ssem0)
    _drain(msg1, gsem1)
    _fire_scatters(ngrp - 1, msg1, ssem1)
    _drain(msg1, ssem1)

    plsc.subcore_barrier()
    pltpu.sync_copy(acc.at[pl.ds(r0, _RPT)], tmp_v)
    pltpu.sync_copy(tmp_v, out_hbm.at[pl.ds(c * _NPAD + r0, _RPT)])


@functools.cache
def _segsum_kernel():
    return pl.kernel(
        _segsum_body,
        out_type=jax.ShapeDtypeStruct((_NCA * _NPAD, _HID), jnp.float32),
        mesh=plsc.VectorSubcoreMesh(core_axis_name="c", subcore_axis_name="s", num_cores=_NCA),
        compiler_params=pltpu.CompilerParams(use_tc_tiling_on_sc=False),
        scratch_types=[
            pltpu.VMEM_SHARED((_NPAD, _HID), jnp.float32),
            pltpu.VMEM_SHARED((_NPAD, _HID), jnp.float32),
            pltpu.VMEM((_GMAX * _K, _CH), jnp.int32),
            pltpu.VMEM((_GMAX * _K, _CH), jnp.int32),
            pltpu.VMEM((_K * _CH, _HID), jnp.float32),
            pltpu.VMEM((_K * _CH, _HID), jnp.float32),
            pltpu.VMEM((_RPT, _HID), jnp.float32),
            pltpu.SemaphoreType.DMA,
            pltpu.SemaphoreType.DMA,
            pltpu.SemaphoreType.DMA,
            pltpu.SemaphoreType.DMA,
        ],
    )


_BLK = 1024
_NBLK = _NPAD // _BLK


def _dinv(degp_ref):
    deg = degp_ref[0, :] + 1.0
    if _NCA == 2:
        deg = deg + degp_ref[1, :]
    return lax.rsqrt(deg)


def _tc1_body(x_ref, w1_ref, degp_ref, out_ref):
    dinv = _dinv(degp_ref)
    h = jnp.dot(x_ref[...], w1_ref[...], preferred_element_type=jnp.float32)
    out_ref[...] = h * dinv[:, None]


def _tc2_body(*refs):
    (*ps, hs1_ref, degp_ref, b1_ref, w2_ref, out_ref) = refs
    dinv = _dinv(degp_ref)
    acc = sum(p[...] for p in ps) + hs1_ref[...]
    acc = acc * dinv[:, None]
    h2 = jnp.maximum(acc + b1_ref[...], 0.0)
    out_ref[...] = jnp.dot(
        h2, w2_ref[...], preferred_element_type=jnp.float32) * dinv[:, None]


def _tc3_body(*refs):
    (*ps, hs2_ref, degp_ref, b2_ref, out_ref) = refs
    dinv = _dinv(degp_ref)
    acc = sum(p[...] for p in ps) + hs2_ref[...]
    acc = acc * dinv[:, None]
    out_ref[...] = jnp.maximum(acc + b2_ref[...], 0.0)


def _row_spec(off):
    return pl.BlockSpec((_BLK, _HID), lambda i, off=off: (i + off, 0))


def kernel(x, edge_index, W1, b1, W2, b2):
    pad = _CALLOC * _CH - _E
    src = jnp.concatenate(
        [edge_index[0], jnp.zeros((pad,), jnp.int32)]).reshape(_CALLOC, _CH)
    dst = jnp.concatenate(
        [edge_index[1], jnp.full((pad,), _N, jnp.int32)]).reshape(
            _CALLOC, _CH)
    x_p = jnp.pad(x, ((0, _NPAD - _N), (0, 0)))
    w2_p = jnp.pad(W2, ((0, 0), (0, _HID - _C)))
    b1_r = b1.reshape(1, _HID)
    b2_r = jnp.pad(b2, (0, _HID - _C)).reshape(1, _HID)

    degp = _deg_kernel()(dst).reshape(_NCA, _NPAD)

    p_specs = [_row_spec(i * _NBLK) for i in range(_NCA)]
    degp_spec = pl.BlockSpec((_NCA, _BLK), lambda i: (0, i))

    hs1 = pl.pallas_call(
        _tc1_body,
        grid=(_NBLK,),
        in_specs=[
            pl.BlockSpec((_BLK, _F_IN), lambda i: (i, 0)),
            pl.BlockSpec((_F_IN, _HID), lambda i: (0, 0)),
            degp_spec,
        ],
        out_specs=pl.BlockSpec((_BLK, _HID), lambda i: (i, 0)),
        out_shape=jax.ShapeDtypeStruct((_NPAD, _HID), jnp.float32),
    )(x_p, W1, degp)

    acc1 = _segsum_kernel()(src, dst, hs1)

    hs2 = pl.pallas_call(
        _tc2_body,
        grid=(_NBLK,),
        in_specs=p_specs + [
            pl.BlockSpec((_BLK, _HID), lambda i: (i, 0)),
            degp_spec,
            pl.BlockSpec((1, _HID), lambda i: (0, 0)),
            pl.BlockSpec((_HID, _HID), lambda i: (0, 0)),
        ],
        out_specs=pl.BlockSpec((_BLK, _HID), lambda i: (i, 0)),
        out_shape=jax.ShapeDtypeStruct((_NPAD, _HID), jnp.float32),
    )(*([acc1] * _NCA), hs1, degp, b1_r, w2_p)

    acc2 = _segsum_kernel()(src, dst, hs2)

    out = pl.pallas_call(
        _tc3_body,
        grid=(_NBLK,),
        in_specs=p_specs + [
            pl.BlockSpec((_BLK, _HID), lambda i: (i, 0)),
            degp_spec,
            pl.BlockSpec((1, _HID), lambda i: (0, 0)),
        ],
        out_specs=pl.BlockSpec((_BLK, _HID), lambda i: (i, 0)),
        out_shape=jax.ShapeDtypeStruct((_NPAD, _HID), jnp.float32),
    )(*([acc2] * _NCA), hs2, degp, b2_r)

    return out[:_N, :_C]


# speedup vs baseline: 1.6581x; 1.0225x over previous
import functools
import jax
import jax.numpy as jnp
from jax import lax
from jax.experimental import pallas as pl
from jax.experimental.pallas import tpu as pltpu
from jax.experimental.pallas import tpu_sc as plsc

_N = 10000
_E = 320000
_F_IN = 128
_HID = 16
_C = 7

_NC = 2
_NS = 16
_CH = 128
_K = 8
_NCA = 2
_G0 = 12
_G1 = 8
_GMAX = max(_G0, _G1)
_NGRP = _NS * (_G0 + _G1)
_NCHK = _NGRP * _K
_EPAD = _NCHK * _CH
_CALLOC = _NCHK + _GMAX * _K
_NPAD = 10240
_RPT = _NPAD // _NS

def _tile_work(c, s):
    if _NCA == 1:
        return _G0, s * (_G0 * _K)
    ngrp = jnp.where(c == 0, _G0, _G1)
    base = jnp.where(c == 0, s * (_G0 * _K),
                     _NS * (_G0 * _K) + s * (_G1 * _K))
    return ngrp, base


def _preload(c, base, hbm, vmem):
    @pl.when(c == 0)
    def _():
        pltpu.sync_copy(hbm.at[pl.ds(base, _G0 * _K)],
                        vmem.at[pl.ds(0, _G0 * _K)])
    @pl.when(c != 0)
    def _():
        pltpu.sync_copy(hbm.at[pl.ds(base, _G1 * _K)],
                        vmem.at[pl.ds(0, _G1 * _K)])


def _deg_body(dst_hbm, out_hbm, accd, idx_d, ones_v, tmp_v, sem):
    c = lax.axis_index("c")
    s = lax.axis_index("s")
    ngrp, base = _tile_work(c, s)

    _preload(c, base, dst_hbm, idx_d)

    def _o(i, _):
        ones_v[pl.ds(i * 16, 16)] = jnp.ones((16,), jnp.float32)
        return 0
    lax.fori_loop(0, _K * _CH // 16, _o, 0)
    def _z(i, _):
        tmp_v[pl.ds(i * 16, 16)] = jnp.zeros((16,), jnp.float32)
        return 0
    lax.fori_loop(0, _RPT // 16, _z, 0)

    r0 = s * _RPT
    pltpu.sync_copy(tmp_v, accd.at[pl.ds(r0, _RPT)])
    plsc.subcore_barrier()

    def _grp(t, _):
        for i in range(_K):
            pltpu.async_copy(ones_v.at[pl.ds(0, _CH)],
                             accd.at[idx_d.at[t * _K + i]], sem, add=True)
        pltpu.make_async_copy(ones_v, accd.at[pl.ds(0, _K * _CH)], sem).wait()
        return 0

    lax.fori_loop(0, ngrp, _grp, 0)
    plsc.subcore_barrier()

    pltpu.sync_copy(accd.at[pl.ds(r0, _RPT)], tmp_v)
    pltpu.sync_copy(tmp_v, out_hbm.at[pl.ds(c * _NPAD + r0, _RPT)])


@functools.cache
def _deg_kernel():
    return pl.kernel(
        _deg_body,
        out_type=jax.ShapeDtypeStruct((_NCA * _NPAD,), jnp.float32),
        mesh=plsc.VectorSubcoreMesh(core_axis_name="c", subcore_axis_name="s", num_cores=_NCA),
        compiler_params=pltpu.CompilerParams(use_tc_tiling_on_sc=False),
        scratch_types=[
            pltpu.VMEM_SHARED((_NPAD,), jnp.float32),
            pltpu.VMEM((_GMAX * _K, _CH), jnp.int32),
            pltpu.VMEM((_K * _CH,), jnp.float32),
            pltpu.VMEM((_RPT,), jnp.float32),
            pltpu.SemaphoreType.DMA,
        ],
    )


def _segsum_body(src_hbm, dst_hbm, hs_hbm, out_hbm, acc, hs_sh, idx_s, idx_d,
                 msg0, msg1, tmp_v, gsem0, gsem1, ssem0, ssem1):
    c = lax.axis_index("c")
    s = lax.axis_index("s")
    ngrp, base = _tile_work(c, s)

    _preload(c, base, src_hbm, idx_s)
    _preload(c, base, dst_hbm, idx_d)

    r0 = s * _RPT
    pltpu.sync_copy(hs_hbm.at[pl.ds(r0, _RPT)], tmp_v)
    pltpu.sync_copy(tmp_v, hs_sh.at[pl.ds(r0, _RPT)])

    def _z(i, _):
        tmp_v[i] = jnp.zeros((16,), jnp.float32)
        return 0
    lax.fori_loop(0, _RPT // 4, _z, 0)
    for q in range(4):
        pltpu.sync_copy(tmp_v.at[pl.ds(0, _RPT // 4)],
                        acc.at[pl.ds(r0 + q * (_RPT // 4), _RPT // 4)])
    plsc.subcore_barrier()

    def _fire_gathers(g, msg, sem):
        for i in range(_K):
            pltpu.async_copy(hs_sh.at[idx_s.at[g * _K + i]],
                             msg.at[pl.ds(i * _CH, _CH)], sem)

    def _fire_scatters(g, msg, sem):
        for i in range(_K):
            pltpu.async_copy(msg.at[pl.ds(i * _CH, _CH)],
                             acc.at[idx_d.at[g * _K + i]], sem, add=True)

    def _drain(msg, sem):
        pltpu.make_async_copy(msg, acc.at[pl.ds(0, _K * _CH)], sem).wait()

    _fire_gathers(0, msg0, gsem0)
    _fire_gathers(1, msg1, gsem1)
    _drain(msg0, gsem0)
    _fire_scatters(0, msg0, ssem0)

    def _pair(t, _):
        g = 2 * t + 1
        _drain(msg0, ssem0)
        _fire_gathers(g + 1, msg0, gsem0)
        _drain(msg1, gsem1)
        _fire_scatters(g, msg1, ssem1)
        _drain(msg1, ssem1)
        _fire_gathers(g + 2, msg1, gsem1)
        _drain(msg0, gsem0)
        _fire_scatters(g + 1, msg0, ssem0)
        return 0

    lax.fori_loop(0, ngrp // 2 - 1, _pair, 0)
    _drain(msg0, ssem0)
    _drain(msg1, gsem1)
    _fire_scatters(ngrp - 1, msg1, ssem1)
    _drain(msg1, ssem1)

    plsc.subcore_barrier()
    pltpu.sync_copy(acc.at[pl.ds(r0, _RPT)], tmp_v)
    pltpu.sync_copy(tmp_v, out_hbm.at[pl.ds(c * _NPAD + r0, _RPT)])


@functools.cache
def _segsum_kernel():
    return pl.kernel(
        _segsum_body,
        out_type=jax.ShapeDtypeStruct((_NCA * _NPAD, _HID), jnp.float32),
        mesh=plsc.VectorSubcoreMesh(core_axis_name="c", subcore_axis_name="s", num_cores=_NCA),
        compiler_params=pltpu.CompilerParams(use_tc_tiling_on_sc=False),
        scratch_types=[
            pltpu.VMEM_SHARED((_NPAD, _HID), jnp.float32),
            pltpu.VMEM_SHARED((_NPAD, _HID), jnp.float32),
            pltpu.VMEM((_GMAX * _K, _CH), jnp.int32),
            pltpu.VMEM((_GMAX * _K, _CH), jnp.int32),
            pltpu.VMEM((_K * _CH, _HID), jnp.float32),
            pltpu.VMEM((_K * _CH, _HID), jnp.float32),
            pltpu.VMEM((_RPT, _HID), jnp.float32),
            pltpu.SemaphoreType.DMA,
            pltpu.SemaphoreType.DMA,
            pltpu.SemaphoreType.DMA,
            pltpu.SemaphoreType.DMA,
        ],
    )


_BLK = 1024
_NBLK = _NPAD // _BLK


def _dinv(degp_ref):
    deg = degp_ref[0, :] + 1.0
    if _NCA == 2:
        deg = deg + degp_ref[1, :]
    return lax.rsqrt(deg)


def _tc1_body(x_ref, w1_ref, degp_ref, out_ref):
    dinv = _dinv(degp_ref)
    h = jnp.dot(x_ref[...], w1_ref[...], preferred_element_type=jnp.float32)
    out_ref[...] = h * dinv[:, None]


def _tc2_body(*refs):
    (*ps, hs1_ref, degp_ref, b1_ref, w2_ref, out_ref) = refs
    dinv = _dinv(degp_ref)
    acc = sum(p[...] for p in ps) + hs1_ref[...]
    acc = acc * dinv[:, None]
    h2 = jnp.maximum(acc + b1_ref[...], 0.0)
    out_ref[...] = jnp.dot(
        h2, w2_ref[...], preferred_element_type=jnp.float32) * dinv[:, None]


def _tc3_body(*refs):
    (*ps, hs2_ref, degp_ref, b2_ref, out_ref) = refs
    dinv = _dinv(degp_ref)
    acc = sum(p[...] for p in ps) + hs2_ref[...]
    acc = acc * dinv[:, None]
    out_ref[...] = jnp.maximum(acc + b2_ref[...], 0.0)


def _row_spec(off):
    return pl.BlockSpec((_BLK, _HID), lambda i, off=off: (i + off, 0))


def kernel(x, edge_index, W1, b1, W2, b2):
    pad = _CALLOC * _CH - _E
    src = jnp.concatenate(
        [edge_index[0], jnp.zeros((pad,), jnp.int32)]).reshape(_CALLOC, _CH)
    dst = jnp.concatenate(
        [edge_index[1], jnp.full((pad,), _N, jnp.int32)]).reshape(
            _CALLOC, _CH)
    x_p = jnp.pad(x, ((0, _NPAD - _N), (0, 0)))
    w2_p = jnp.pad(W2, ((0, 0), (0, _HID - _C)))
    b1_r = b1.reshape(1, _HID)
    b2_r = jnp.pad(b2, (0, _HID - _C)).reshape(1, _HID)

    degp = _deg_kernel()(dst).reshape(_NCA, _NPAD)

    p_specs = [_row_spec(i * _NBLK) for i in range(_NCA)]
    degp_spec = pl.BlockSpec((_NCA, _BLK), lambda i: (0, i))

    hs1 = pl.pallas_call(
        _tc1_body,
        grid=(_NBLK,),
        in_specs=[
            pl.BlockSpec((_BLK, _F_IN), lambda i: (i, 0)),
            pl.BlockSpec((_F_IN, _HID), lambda i: (0, 0)),
            degp_spec,
        ],
        out_specs=pl.BlockSpec((_BLK, _HID), lambda i: (i, 0)),
        out_shape=jax.ShapeDtypeStruct((_NPAD, _HID), jnp.float32),
    )(x_p, W1, degp)

    acc1 = _segsum_kernel()(src, dst, hs1)

    hs2 = pl.pallas_call(
        _tc2_body,
        grid=(_NBLK,),
        in_specs=p_specs + [
            pl.BlockSpec((_BLK, _HID), lambda i: (i, 0)),
            degp_spec,
            pl.BlockSpec((1, _HID), lambda i: (0, 0)),
            pl.BlockSpec((_HID, _HID), lambda i: (0, 0)),
        ],
        out_specs=pl.BlockSpec((_BLK, _HID), lambda i: (i, 0)),
        out_shape=jax.ShapeDtypeStruct((_NPAD, _HID), jnp.float32),
    )(*([acc1] * _NCA), hs1, degp, b1_r, w2_p)

    acc2 = _segsum_kernel()(src, dst, hs2)

    out = pl.pallas_call(
        _tc3_body,
        grid=(_NBLK,),
        in_specs=p_specs + [
            pl.BlockSpec((_BLK, _HID), lambda i: (i, 0)),
            degp_spec,
            pl.BlockSpec((1, _HID), lambda i: (0, 0)),
        ],
        out_specs=pl.BlockSpec((_BLK, _HID), lambda i: (i, 0)),
        out_shape=jax.ShapeDtypeStruct((_NPAD, _HID), jnp.float32),
    )(*([acc2] * _NCA), hs2, degp, b2_r)

    return out[:_N, :_C]
